# baseline (speedup 1.0000x reference)
import functools

import jax
import jax.numpy as jnp
from jax import lax
from jax.experimental import pallas as pl
from jax.experimental.pallas import tpu as pltpu
from jax.experimental.pallas import tpu_sc as plsc

_VOCAB = 1_000_000
_HID = 64
_NCLS = 4
_CPAD = 8
_VBLK = 10_000

_NTOK = 16384 * 200


def _proj_body(table_ref, w_ref, b_ref, out_ref):
    out_ref[...] = lax.dot_general(
        table_ref[...], w_ref[...],
        dimension_numbers=(((1,), (1,)), ((), ())),
        preferred_element_type=jnp.float32,
        precision=lax.Precision.HIGHEST,
    ) + b_ref[...]


def _project_table(table, w_pad, b_pad):
    grid = _VOCAB // _VBLK
    return pl.pallas_call(
        _proj_body,
        grid=(grid,),
        in_specs=[
            pl.BlockSpec((_VBLK, _HID), lambda i: (i, 0)),
            pl.BlockSpec((_CPAD, _HID), lambda i: (0, 0)),
            pl.BlockSpec((1, _CPAD), lambda i: (0, 0)),
        ],
        out_specs=pl.BlockSpec((_VBLK, _CPAD), lambda i: (i, 0)),
        out_shape=jax.ShapeDtypeStruct((_VOCAB, _CPAD), jnp.float32),
    )(table, w_pad, b_pad)


@functools.lru_cache(maxsize=1)
def _make_gather():
    nc, ns = 2, 16
    nw = nc * ns
    b_per_w = _NTOK // nw
    chunk = 10_240
    n_chunks = b_per_w // chunk

    mesh = plsc.VectorSubcoreMesh(core_axis_name="c", subcore_axis_name="s")

    @functools.partial(
        pl.kernel,
        mesh=mesh,
        out_type=jax.ShapeDtypeStruct((_NTOK, _CPAD), jnp.float32),
        scratch_types=[
            pltpu.VMEM((chunk,), jnp.int32),
            pltpu.VMEM((chunk, _CPAD), jnp.float32),
            pltpu.SemaphoreType.DMA,
        ],
        compiler_params=pltpu.CompilerParams(use_tc_tiling_on_sc=False),
    )
    def gather_kernel(idx_hbm, proj_hbm, out_hbm, idx_v, rows_v, sem):
        wid = lax.axis_index("s") * nc + lax.axis_index("c")
        base = wid * b_per_w

        def body(i, carry):
            off = base + i * chunk
            pltpu.sync_copy(idx_hbm.at[pl.ds(off, chunk)], idx_v)
            pltpu.async_copy(proj_hbm.at[idx_v], rows_v, sem).wait()
            pltpu.sync_copy(rows_v, out_hbm.at[pl.ds(off, chunk)])
            return carry

        lax.fori_loop(0, n_chunks, body, 0)

    return gather_kernel


def kernel(inputs, table, W, b):
    w_pad = jnp.zeros((_CPAD, _HID), jnp.float32).at[:_NCLS].set(W)
    b_pad = jnp.zeros((1, _CPAD), jnp.float32).at[0, :_NCLS].set(b)
    proj = _project_table(table, w_pad, b_pad)
    flat_idx = inputs.reshape(-1).astype(jnp.int32)
    out = _make_gather()(flat_idx, proj)
    return out[:, :_NCLS].reshape(inputs.shape[0], inputs.shape[1], _NCLS)


# speedup vs baseline: 1.0994x; 1.0994x over previous
import functools

import jax
import jax.numpy as jnp
from jax import lax
from jax.experimental import pallas as pl
from jax.experimental.pallas import tpu as pltpu
from jax.experimental.pallas import tpu_sc as plsc

_VOCAB = 1_000_000
_HID = 64
_NCLS = 4
_CPAD = 8
_PACK = 16
_NROW = _VOCAB // _PACK
_KDIM = _PACK * _HID
_LANE = _PACK * _CPAD
_RBLK = 512

_NTOK = 16384 * 200


def _proj_body(tab_ref, wbig_ref, brow_ref, out_ref):
    out_ref[...] = lax.dot_general(
        tab_ref[...], wbig_ref[...],
        dimension_numbers=(((1,), (0,)), ((), ())),
        preferred_element_type=jnp.float32,
        precision=lax.Precision.HIGHEST,
    ) + brow_ref[...]


def _project_table(table_packed, wbig, brow):
    grid = pl.cdiv(_NROW, _RBLK)
    return pl.pallas_call(
        _proj_body,
        grid=(grid,),
        in_specs=[
            pl.BlockSpec((_RBLK, _KDIM), lambda i: (i, 0)),
            pl.BlockSpec((_KDIM, _LANE), lambda i: (0, 0)),
            pl.BlockSpec((1, _LANE), lambda i: (0, 0)),
        ],
        out_specs=pl.BlockSpec((_RBLK, _LANE), lambda i: (i, 0)),
        out_shape=jax.ShapeDtypeStruct((_NROW, _LANE), jnp.float32),
    )(table_packed, wbig, brow)


@functools.lru_cache(maxsize=1)
def _make_gather():
    nc, ns = 2, 16
    nw = nc * ns
    b_per_w = _NTOK // nw
    chunk = 10_240
    n_chunks = b_per_w // chunk

    mesh = plsc.VectorSubcoreMesh(core_axis_name="c", subcore_axis_name="s")

    @functools.partial(
        pl.kernel,
        mesh=mesh,
        out_type=jax.ShapeDtypeStruct((_NTOK, _CPAD), jnp.float32),
        scratch_types=[
            pltpu.VMEM((chunk,), jnp.int32),
            pltpu.VMEM((chunk, _CPAD), jnp.float32),
            pltpu.SemaphoreType.DMA,
        ],
        compiler_params=pltpu.CompilerParams(use_tc_tiling_on_sc=False),
    )
    def gather_kernel(idx_hbm, proj_hbm, out_hbm, idx_v, rows_v, sem):
        wid = lax.axis_index("s") * nc + lax.axis_index("c")
        base = wid * b_per_w

        def body(i, carry):
            off = base + i * chunk
            pltpu.sync_copy(idx_hbm.at[pl.ds(off, chunk)], idx_v)
            pltpu.async_copy(proj_hbm.at[idx_v], rows_v, sem).wait()
            pltpu.sync_copy(rows_v, out_hbm.at[pl.ds(off, chunk)])
            return carry

        lax.fori_loop(0, n_chunks, body, 0)

    return gather_kernel


def kernel(inputs, table, W, b):
    w_pad = jnp.zeros((_CPAD, _HID), jnp.float32).at[:_NCLS].set(W)
    wbig = jnp.einsum('uv,ch->uhvc', jnp.eye(_PACK, dtype=jnp.float32),
                      w_pad).reshape(_KDIM, _LANE)
    b_pad = jnp.zeros((_CPAD,), jnp.float32).at[:_NCLS].set(b)
    brow = jnp.tile(b_pad, _PACK).reshape(1, _LANE)

    table_packed = table.reshape(_NROW, _KDIM)
    proj = _project_table(table_packed, wbig, brow).reshape(_VOCAB, _CPAD)

    flat_idx = inputs.reshape(-1).astype(jnp.int32)
    out = _make_gather()(flat_idx, proj)
    return out[:, :_NCLS].reshape(inputs.shape[0], inputs.shape[1], _NCLS)
